# baseline (speedup 1.0000x reference)
import functools

import jax
import jax.numpy as jnp
from jax import lax
from jax.experimental import pallas as pl
from jax.experimental.pallas import tpu as pltpu
from jax.experimental.pallas import tpu_sc as plsc

NUM_PAGES = 2048
PAGE_SIZE = 64
KV_DIM = 256
MAX_SEQS = 64
PAGES_PER_SEQ = 64
MAX_SEQ_LEN = 4096

_ROW = PAGE_SIZE * KV_DIM
_NW = 32
_PPW = NUM_PAGES // _NW
_CH = 2
_NST = _PPW // _CH
_RPW = MAX_SEQS // _NW


def _sc_body(cache_h, tokens_h, seq_h, pi_h, meta_h,
             cache_o, tokens_o, seq_o, pi_o,
             b0, b1, b2, idxs, ridx, tbuf, pbuf, piv, mv, sv, sov,
             g0, g1, g2, s0, s1, s2):
    bufs = (b0, b1, b2)
    gs = (g0, g1, g2)
    ss = (s0, s1, s2)
    cid = lax.axis_index("c")
    sid = lax.axis_index("s")
    wid = sid * 2 + cid
    base = wid * _PPW

    pltpu.sync_copy(meta_h, mv)
    pltpu.sync_copy(seq_h, sv)
    lanes = lax.iota(jnp.int32, 16)
    zeros16 = jnp.zeros((16,), jnp.int32)
    parent_v = plsc.load_gather(mv, [zeros16 + 1])
    child_v = plsc.load_gather(mv, [zeros16 + 2])
    fresh_v = plsc.load_gather(mv, [zeros16 + 3])

    src_len_v = plsc.load_gather(sv, [parent_v])
    last_v = (src_len_v + PAGE_SIZE - 1) // PAGE_SIZE - 1
    safe_last_v = jnp.maximum(last_v, 0)
    partial_v = jnp.logical_and(src_len_v % PAGE_SIZE != 0, src_len_v > 0)

    pltpu.sync_copy(pi_h, piv)
    src_page_v = plsc.load_gather(piv, [parent_v, safe_last_v])
    dst_page_v = jnp.where(partial_v, fresh_v, src_page_v)

    for j in range(4):
        pages_j = base + lanes + 16 * j
        sel_j = jnp.where(pages_j == dst_page_v, src_page_v, pages_j)
        plsc.store_scatter(idxs, [(lanes + 16 * j) // _CH,
                                  (lanes + 16 * j) % _CH], sel_j)

    def gcopy(c, b):
        return pltpu.make_async_copy(
            cache_h.at[idxs.at[c]], bufs[b], gs[b])

    def scopy(c, b):
        return pltpu.make_async_copy(
            bufs[b], cache_o.at[pl.ds(base + _CH * c, _CH)], ss[b])

    gcopy(0, 0).start()
    for c in range(_NST):
        b = c % 3
        if c + 1 < _NST:
            nb = (c + 1) % 3
            if c - 2 >= 0:
                scopy(c - 2, nb).wait()
            gcopy(c + 1, nb).start()
        gcopy(c, b).wait()
        scopy(c, b).start()
    scopy(_NST - 2, (_NST - 2) % 3).wait()
    scopy(_NST - 1, (_NST - 1) % 3).wait()

    rows_v = wid * _RPW + lanes
    rsel_v = jnp.where(rows_v == child_v, parent_v, rows_v)
    plsc.store_scatter(ridx, [lanes], rsel_v, mask=lanes < _RPW)
    pltpu.sync_copy(tokens_h.at[ridx], tbuf)
    pltpu.sync_copy(tbuf, tokens_o.at[pl.ds(wid * _RPW, _RPW)])

    for r_off in range(_RPW):
        r = wid * _RPW + r_off
        rsrc_v = jnp.where(r == child_v, parent_v, zeros16 + r)
        is_child_v = jnp.logical_and(r == child_v, partial_v)
        for j in range(PAGES_PER_SEQ // 16):
            v = plsc.load_gather(piv, [rsrc_v, lanes + 16 * j])
            mask = jnp.logical_and(lanes + 16 * j == safe_last_v, is_child_v)
            pbuf[r_off, pl.ds(16 * j, 16)] = jnp.where(mask, fresh_v, v)
    pltpu.sync_copy(pbuf, pi_o.at[pl.ds(wid * _RPW, _RPW)])

    @pl.when(wid == 0)
    def _seq():
        for j in range(MAX_SEQS // 16):
            slj = sv[pl.ds(16 * j, 16)]
            sov[pl.ds(16 * j, 16)] = jnp.where(lanes + 16 * j == child_v,
                                               src_len_v, slj)
        pltpu.sync_copy(sov, seq_o)


def kernel(cache, tokens, seq_lens, page_indices, parent_local_id,
           child_local_id, fresh_page):
    meta = jnp.stack([jnp.asarray(parent_local_id, jnp.int32),
                      jnp.asarray(child_local_id, jnp.int32),
                      jnp.asarray(fresh_page, jnp.int32)])
    meta16 = jnp.pad(meta, (1, 12))
    cache2 = cache.reshape(NUM_PAGES, _ROW)

    mesh = plsc.VectorSubcoreMesh(core_axis_name="c", subcore_axis_name="s")
    sck = functools.partial(
        pl.kernel,
        mesh=mesh,
        compiler_params=pltpu.CompilerParams(needs_layout_passes=False),
        out_type=[
            jax.ShapeDtypeStruct((NUM_PAGES, _ROW), cache.dtype),
            jax.ShapeDtypeStruct(tokens.shape, tokens.dtype),
            jax.ShapeDtypeStruct(seq_lens.shape, seq_lens.dtype),
            jax.ShapeDtypeStruct(page_indices.shape, page_indices.dtype),
        ],
        scratch_types=[
            pltpu.VMEM((_CH, _ROW), jnp.float32),
            pltpu.VMEM((_CH, _ROW), jnp.float32),
            pltpu.VMEM((_CH, _ROW), jnp.float32),
            pltpu.VMEM((_NST, _CH), jnp.int32),
            pltpu.VMEM((_RPW,), jnp.int32),
            pltpu.VMEM((_RPW, MAX_SEQ_LEN), jnp.int32),
            pltpu.VMEM((_RPW, PAGES_PER_SEQ), jnp.int32),
            pltpu.VMEM((MAX_SEQS, PAGES_PER_SEQ), jnp.int32),
            pltpu.VMEM((16,), jnp.int32),
            pltpu.VMEM((MAX_SEQS,), jnp.int32),
            pltpu.VMEM((MAX_SEQS,), jnp.int32),
            pltpu.SemaphoreType.DMA,
            pltpu.SemaphoreType.DMA,
            pltpu.SemaphoreType.DMA,
            pltpu.SemaphoreType.DMA,
            pltpu.SemaphoreType.DMA,
            pltpu.SemaphoreType.DMA,
        ],
    )(_sc_body)
    cache_out, tokens_out, seq_lens_out, pi_out = sck(
        cache2, tokens, seq_lens, page_indices, meta16)
    return (cache_out.reshape(NUM_PAGES, PAGE_SIZE, KV_DIM),
            tokens_out, seq_lens_out, pi_out)


# speedup vs baseline: 2.9237x; 2.9237x over previous
import functools

import jax
import jax.numpy as jnp
from jax import lax
from jax.experimental import pallas as pl
from jax.experimental.pallas import tpu as pltpu
from jax.experimental.pallas import tpu_sc as plsc

NUM_PAGES = 2048
PAGE_SIZE = 64
KV_DIM = 256
MAX_SEQS = 64
PAGES_PER_SEQ = 64
MAX_SEQ_LEN = 4096


_BP = 32
_NCH = NUM_PAGES // _BP
_L = 8
_M = 8
_K = _L + _M


def _tc_body(meta_s, seq_lens_s, pi_s, cache_a, cache_out_a,
             bufs, pbuf, in_sems, out_sems, psem):

    def in_copy(c):
        k = jax.lax.rem(c, _K)
        return pltpu.make_async_copy(
            cache_a.at[pl.ds(c * _BP, _BP)], bufs.at[k], in_sems.at[k])

    def out_copy(c):
        k = jax.lax.rem(c, _K)
        return pltpu.make_async_copy(
            bufs.at[k], cache_out_a.at[pl.ds(c * _BP, _BP)], out_sems.at[k])

    for c in range(_L):
        in_copy(c).start()

    def step(c, carry):
        @pl.when(c >= _M)
        def _():
            out_copy(c - _M).wait()

        @pl.when(c + _L < _NCH)
        def _():
            in_copy(c + _L).start()

        in_copy(c).wait()
        out_copy(c).start()
        return carry

    jax.lax.fori_loop(0, _NCH, step, 0)

    parent = meta_s[0]
    fresh = meta_s[1]
    src_len = seq_lens_s[parent]
    safe_last = jnp.maximum((src_len + PAGE_SIZE - 1) // PAGE_SIZE - 1, 0)
    has_partial = jnp.logical_and(src_len % PAGE_SIZE != 0, src_len > 0)
    src_page = pi_s[parent, safe_last]
    dst_page = jnp.where(has_partial, fresh, src_page)

    def drain(c, carry):
        out_copy(c).wait()
        return carry
    jax.lax.fori_loop(max(_NCH - _M, 0), _NCH, drain, 0)

    fin = pltpu.make_async_copy(cache_a.at[pl.ds(src_page, 1)], pbuf, psem)
    fin.start()
    fin.wait()
    fout = pltpu.make_async_copy(pbuf, cache_out_a.at[pl.ds(dst_page, 1)], psem)
    fout.start()
    fout.wait()


def _clone_cache(cache, meta, seq_lens, page_indices):
    return pl.pallas_call(
        _tc_body,
        in_specs=[
            pl.BlockSpec(memory_space=pltpu.SMEM),
            pl.BlockSpec(memory_space=pltpu.SMEM),
            pl.BlockSpec(memory_space=pltpu.SMEM),
            pl.BlockSpec(memory_space=pl.ANY),
        ],
        out_specs=pl.BlockSpec(memory_space=pl.ANY),
        out_shape=jax.ShapeDtypeStruct(cache.shape, cache.dtype),
        scratch_shapes=[
            pltpu.VMEM((_K, _BP, PAGE_SIZE, KV_DIM), jnp.float32),
            pltpu.VMEM((1, PAGE_SIZE, KV_DIM), jnp.float32),
            pltpu.SemaphoreType.DMA((_K,)),
            pltpu.SemaphoreType.DMA((_K,)),
            pltpu.SemaphoreType.DMA,
        ],
        compiler_params=pltpu.CompilerParams(
            vmem_limit_bytes=128 * 1024 * 1024),
    )(meta, seq_lens, page_indices, cache)


_NW = 32
_RPW = MAX_SEQS // _NW


def _sc_body(tokens_h, seq_h, pi_h, meta_h,
             tokens_o, seq_o, pi_o,
             ridx, tbuf, pbuf, piv, mv, sv, sov):
    cid = lax.axis_index("c")
    sid = lax.axis_index("s")
    wid = sid * 2 + cid

    pltpu.sync_copy(meta_h, mv)
    pltpu.sync_copy(seq_h, sv)
    pltpu.sync_copy(pi_h, piv)
    lanes = lax.iota(jnp.int32, 16)
    zeros16 = jnp.zeros((16,), jnp.int32)
    parent_v = plsc.load_gather(mv, [zeros16 + 1])
    child_v = plsc.load_gather(mv, [zeros16 + 2])
    fresh_v = plsc.load_gather(mv, [zeros16 + 3])

    src_len_v = plsc.load_gather(sv, [parent_v])
    last_v = (src_len_v + PAGE_SIZE - 1) // PAGE_SIZE - 1
    safe_last_v = jnp.maximum(last_v, 0)
    partial_v = jnp.logical_and(src_len_v % PAGE_SIZE != 0, src_len_v > 0)

    rows_v = wid * _RPW + lanes
    rsel_v = jnp.where(rows_v == child_v, parent_v, rows_v)
    plsc.store_scatter(ridx, [lanes], rsel_v, mask=lanes < _RPW)
    pltpu.sync_copy(tokens_h.at[ridx], tbuf)
    pltpu.sync_copy(tbuf, tokens_o.at[pl.ds(wid * _RPW, _RPW)])

    for r_off in range(_RPW):
        r = wid * _RPW + r_off
        rsrc_v = jnp.where(r == child_v, parent_v, zeros16 + r)
        is_child_v = jnp.logical_and(r == child_v, partial_v)
        for j in range(PAGES_PER_SEQ // 16):
            v = plsc.load_gather(piv, [rsrc_v, lanes + 16 * j])
            mask = jnp.logical_and(lanes + 16 * j == safe_last_v, is_child_v)
            pbuf[r_off, pl.ds(16 * j, 16)] = jnp.where(mask, fresh_v, v)
    pltpu.sync_copy(pbuf, pi_o.at[pl.ds(wid * _RPW, _RPW)])

    @pl.when(wid == 0)
    def _seq():
        for j in range(MAX_SEQS // 16):
            slj = sv[pl.ds(16 * j, 16)]
            sov[pl.ds(16 * j, 16)] = jnp.where(lanes + 16 * j == child_v,
                                               src_len_v, slj)
        pltpu.sync_copy(sov, seq_o)


def _clone_metadata(tokens, seq_lens, page_indices, meta16):
    mesh = plsc.VectorSubcoreMesh(core_axis_name="c", subcore_axis_name="s")
    sck = functools.partial(
        pl.kernel,
        mesh=mesh,
        compiler_params=pltpu.CompilerParams(needs_layout_passes=False),
        out_type=[
            jax.ShapeDtypeStruct(tokens.shape, tokens.dtype),
            jax.ShapeDtypeStruct(seq_lens.shape, seq_lens.dtype),
            jax.ShapeDtypeStruct(page_indices.shape, page_indices.dtype),
        ],
        scratch_types=[
            pltpu.VMEM((_RPW,), jnp.int32),
            pltpu.VMEM((_RPW, MAX_SEQ_LEN), jnp.int32),
            pltpu.VMEM((_RPW, PAGES_PER_SEQ), jnp.int32),
            pltpu.VMEM((MAX_SEQS, PAGES_PER_SEQ), jnp.int32),
            pltpu.VMEM((16,), jnp.int32),
            pltpu.VMEM((MAX_SEQS,), jnp.int32),
            pltpu.VMEM((MAX_SEQS,), jnp.int32),
        ],
    )(_sc_body)
    return sck(tokens, seq_lens, page_indices, meta16)


def kernel(cache, tokens, seq_lens, page_indices, parent_local_id,
           child_local_id, fresh_page):
    parent = jnp.asarray(parent_local_id, jnp.int32)
    child = jnp.asarray(child_local_id, jnp.int32)
    fresh = jnp.asarray(fresh_page, jnp.int32)
    meta_tc = jnp.stack([parent, fresh])
    meta16 = jnp.pad(jnp.stack([parent, child, fresh]), (1, 12))

    cache_out = _clone_cache(cache, meta_tc, seq_lens, page_indices)
    tokens_out, seq_lens_out, pi_out = _clone_metadata(
        tokens, seq_lens, page_indices, meta16)
    return (cache_out, tokens_out, seq_lens_out, pi_out)
